# speedup vs baseline: 1.1510x; 1.1510x over previous
import jax
import jax.numpy as jnp
from jax import lax
from jax.experimental import pallas as pl
from jax.experimental.pallas import tpu as pltpu

VOCAB = 1000
ROWS_PER_BLOCK = 2048


def _one_hot_block(arr_ref, out_ref):
    ids = arr_ref[...]
    iota = lax.broadcasted_iota(jnp.int32, out_ref.shape, 1)
    out_ref[...] = (ids == iota).astype(jnp.float32)


def kernel(arr, mask):
    del mask
    b, h = arr.shape
    n = b * h
    flat = arr.reshape(n, 1).astype(jnp.int32)
    r = ROWS_PER_BLOCK
    out = pl.pallas_call(
        _one_hot_block,
        grid=(n // r,),
        in_specs=[pl.BlockSpec((r, 1), lambda i: (i, 0))],
        out_specs=pl.BlockSpec((r, VOCAB), lambda i: (i, 0)),
        out_shape=jax.ShapeDtypeStruct((n, VOCAB), jnp.float32),
        compiler_params=pltpu.CompilerParams(
            dimension_semantics=("parallel",),
        ),
    )(flat)
    return out.reshape(b, h, VOCAB)


# speedup vs baseline: 1.8801x; 1.6334x over previous
import jax
import jax.numpy as jnp
from jax import lax
from jax.experimental import pallas as pl
from jax.experimental.pallas import tpu as pltpu

VOCAB = 1000
BATCH_BLOCK = 128


def _one_hot_block(arr_ref, out_ref):
    ids = arr_ref[...]
    iota = lax.broadcasted_iota(jnp.int32, out_ref.shape, 2)
    out_ref[...] = (ids[:, :, None] == iota).astype(jnp.float32)


def kernel(arr, mask):
    del mask
    b, h = arr.shape
    r = BATCH_BLOCK
    return pl.pallas_call(
        _one_hot_block,
        grid=(b // r,),
        in_specs=[pl.BlockSpec((r, h), lambda i: (i, 0))],
        out_specs=pl.BlockSpec((r, h, VOCAB), lambda i: (i, 0, 0)),
        out_shape=jax.ShapeDtypeStruct((b, h, VOCAB), jnp.float32),
        compiler_params=pltpu.CompilerParams(
            dimension_semantics=("parallel",),
        ),
    )(arr.astype(jnp.int32))
